# baseline (speedup 1.0000x reference)
import functools

import jax
import jax.numpy as jnp
from jax import lax
from jax.experimental import pallas as pl
from jax.experimental.pallas import tpu as pltpu
from jax.experimental.pallas import tpu_sc as plsc

T_, D_ = 2048, 768
E_, FF_ = 8, 3072
P_ = 2 * T_
TM = 256
NT_ = P_ // TM + E_
RPAD_ = NT_ * TM
RB_ = 512
LC_ = D_ // 16

NC_, NS_ = 2, 16
NW_ = NC_ * NS_
CH_ = P_ // NW_
TT_ = T_ // NW_

_INV_SQRT2 = 0.7071067811865476


def _router_meta_body(x_ref, wg_ref, s_ref, s0_ref, s1_ref, w0_ref, w1_ref,
                      te_ref):
    xt = x_ref[...]
    logits = lax.dot_general(xt, wg_ref[...], (((1,), (1,)), ((), ())),
                             preferred_element_type=jnp.float32)
    lane = lax.broadcasted_iota(jnp.int32, (T_, E_), 1)
    mx = jnp.max(logits, axis=1, keepdims=True)
    p = jnp.exp(logits - mx)
    p = p / jnp.sum(p, axis=1, keepdims=True)
    m1 = jnp.max(p, axis=1, keepdims=True)
    e1 = jnp.min(jnp.where(p == m1, lane, E_), axis=1, keepdims=True)
    pm = jnp.where(lane == e1, -1.0, p)
    m2 = jnp.max(pm, axis=1, keepdims=True)
    e2 = jnp.min(jnp.where(pm == m2, lane, E_), axis=1, keepdims=True)
    ssum = m1 + m2
    w0_ref[...] = m1 / ssum
    w1_ref[...] = m2 / ssum

    lane_p = lax.broadcasted_iota(jnp.int32, (P_, E_), 1)
    e_pair = jnp.concatenate([e1, e2], axis=0)
    onehot = (e_pair == lane_p).astype(jnp.float32)

    r_i = lax.broadcasted_iota(jnp.int32, (RB_, RB_), 0)
    c_i = lax.broadcasted_iota(jnp.int32, (RB_, RB_), 1)
    ltri = (r_i > c_i).astype(jnp.float32)
    carry = jnp.zeros((1, E_), jnp.float32)
    rank_blocks = []
    for b in range(P_ // RB_):
        mb = onehot[b * RB_:(b + 1) * RB_]
        rb = lax.dot_general(ltri, mb, (((1,), (0,)), ((), ())),
                             preferred_element_type=jnp.float32) + carry
        rank_blocks.append(rb)
        carry = carry + jnp.sum(mb, axis=0, keepdims=True)
    ranks = jnp.concatenate(rank_blocks, axis=0)
    counts = carry

    padded = jnp.floor((counts + (TM - 1)) / TM) * TM
    e8r = lax.broadcasted_iota(jnp.int32, (E_, E_), 0)
    e8c = lax.broadcasted_iota(jnp.int32, (E_, E_), 1)
    ustri = (e8r < e8c).astype(jnp.float32)
    pad_off = lax.dot_general(padded, ustri, (((1,), (0,)), ((), ())),
                              preferred_element_type=jnp.float32)
    slots = jnp.sum(onehot * (ranks + pad_off), axis=1, keepdims=True)
    slots_i = slots.astype(jnp.int32)
    s_ref[...] = slots_i
    s0_ref[...] = slots_i[:T_]
    s1_ref[...] = slots_i[T_:]

    end_t = (pad_off + padded) / TM
    total_t = jnp.sum(padded, axis=1, keepdims=True) / TM
    ti = lax.broadcasted_iota(jnp.int32, (NT_, E_), 0).astype(jnp.float32)
    n_done = jnp.sum(jnp.where(ti >= end_t, 1.0, 0.0), axis=1, keepdims=True)
    expert_col = jnp.minimum(n_done, float(E_ - 1))
    active_col = jnp.where(ti[:, :1] < total_t, 1.0, 0.0)
    te_ref[...] = jnp.concatenate([expert_col, active_col],
                                  axis=1).astype(jnp.int32)


def _router_meta(x_flat, Wg):
    return pl.pallas_call(
        _router_meta_body,
        out_shape=(
            jax.ShapeDtypeStruct((P_, 1), jnp.int32),
            jax.ShapeDtypeStruct((T_, 1), jnp.int32),
            jax.ShapeDtypeStruct((T_, 1), jnp.int32),
            jax.ShapeDtypeStruct((T_, 1), jnp.float32),
            jax.ShapeDtypeStruct((T_, 1), jnp.float32),
            jax.ShapeDtypeStruct((NT_, 2), jnp.int32),
        ),
    )(x_flat, Wg)


@functools.lru_cache(maxsize=None)
def _sc_kernels():
    mesh = plsc.VectorSubcoreMesh(core_axis_name="c", subcore_axis_name="s")

    @functools.partial(
        pl.kernel,
        out_type=jax.ShapeDtypeStruct((RPAD_, D_), jnp.float32),
        mesh=mesh,
        scratch_types=[
            pltpu.VMEM((CH_,), jnp.int32),
            pltpu.VMEM((CH_, D_), jnp.float32),
            pltpu.SemaphoreType.DMA,
        ],
    )
    def _dispatch(x_hbm, s_hbm, xs_hbm, idx_v, rows_v, sem):
        wid = lax.axis_index("s") * NC_ + lax.axis_index("c")
        base = wid * CH_
        tbase = lax.rem(base, T_)
        pltpu.sync_copy(s_hbm.at[pl.ds(base, CH_)], idx_v)
        pltpu.sync_copy(x_hbm.at[pl.ds(tbase, CH_)], rows_v)
        pltpu.async_copy(rows_v, xs_hbm.at[idx_v], sem).wait()

    @functools.partial(
        pl.kernel,
        out_type=jax.ShapeDtypeStruct((T_, D_), jnp.float32),
        mesh=mesh,
        scratch_types=[
            pltpu.VMEM((TT_,), jnp.int32),
            pltpu.VMEM((TT_,), jnp.int32),
            pltpu.VMEM((TT_ + 16,), jnp.float32),
            pltpu.VMEM((TT_ + 16,), jnp.float32),
            pltpu.VMEM((TT_, D_), jnp.float32),
            pltpu.VMEM((TT_, D_), jnp.float32),
            pltpu.SemaphoreType.DMA,
        ],
    )
    def _combine(ys_hbm, s0_hbm, s1_hbm, w0_hbm, w1_hbm, out_hbm,
                 i0_v, i1_v, w0_v, w1_v, r0_v, r1_v, sem):
        wid = lax.axis_index("s") * NC_ + lax.axis_index("c")
        tb = wid * TT_
        pltpu.sync_copy(s0_hbm.at[pl.ds(tb, TT_)], i0_v)
        pltpu.sync_copy(s1_hbm.at[pl.ds(tb, TT_)], i1_v)
        pltpu.sync_copy(w0_hbm.at[pl.ds(tb, TT_)], w0_v.at[pl.ds(0, TT_)])
        pltpu.sync_copy(w1_hbm.at[pl.ds(tb, TT_)], w1_v.at[pl.ds(0, TT_)])
        pltpu.async_copy(ys_hbm.at[i0_v], r0_v, sem).wait()
        pltpu.async_copy(ys_hbm.at[i1_v], r1_v, sem).wait()

        def body(i, _):
            a = w0_v[pl.ds(i, 16)][0]
            b = w1_v[pl.ds(i, 16)][0]
            for c in range(LC_):
                sl = pl.ds(c * 16, 16)
                r0_v[i, sl] = a * r0_v[i, sl] + b * r1_v[i, sl]
            return 0

        lax.fori_loop(0, TT_, body, 0)
        pltpu.sync_copy(r0_v, out_hbm.at[pl.ds(tb, TT_)])

    return _dispatch, _combine


FH_ = FF_ // 2


def _expert_mlp_body(te_ref, xs_ref, w1a_ref, w1b_ref, w2a_ref, w2b_ref,
                     ys_ref):
    i = pl.program_id(0)

    @pl.when(te_ref[i, 1] == 1)
    def _():
        xb = xs_ref[...].astype(jnp.bfloat16)
        dn = (((1,), (0,)), ((), ()))
        h1 = lax.dot_general(xb, w1a_ref[0].astype(jnp.bfloat16), dn,
                             preferred_element_type=jnp.float32)
        h2 = lax.dot_general(xb, w1b_ref[0].astype(jnp.bfloat16), dn,
                             preferred_element_type=jnp.float32)
        g1 = (0.5 * h1 * (1.0 + lax.erf(h1 * _INV_SQRT2))).astype(jnp.bfloat16)
        g2 = (0.5 * h2 * (1.0 + lax.erf(h2 * _INV_SQRT2))).astype(jnp.bfloat16)
        y1 = lax.dot_general(g1, w2a_ref[0].astype(jnp.bfloat16), dn,
                             preferred_element_type=jnp.float32)
        y2 = lax.dot_general(g2, w2b_ref[0].astype(jnp.bfloat16), dn,
                             preferred_element_type=jnp.float32)
        ys_ref[...] = y1 + y2


def _expert_mlp(te, xs, W1, W2):
    grid_spec = pltpu.PrefetchScalarGridSpec(
        num_scalar_prefetch=1,
        grid=(NT_,),
        in_specs=[
            pl.BlockSpec((TM, D_), lambda i, te: (i, 0)),
            pl.BlockSpec((1, D_, FH_), lambda i, te: (te[i, 0], 0, 0)),
            pl.BlockSpec((1, D_, FH_), lambda i, te: (te[i, 0], 0, 1)),
            pl.BlockSpec((1, FH_, D_), lambda i, te: (te[i, 0], 0, 0)),
            pl.BlockSpec((1, FH_, D_), lambda i, te: (te[i, 0], 1, 0)),
        ],
        out_specs=pl.BlockSpec((TM, D_), lambda i, te: (i, 0)),
    )
    return pl.pallas_call(
        _expert_mlp_body,
        grid_spec=grid_spec,
        out_shape=jax.ShapeDtypeStruct((RPAD_, D_), jnp.float32),
        compiler_params=pltpu.CompilerParams(
            dimension_semantics=("arbitrary",),
            vmem_limit_bytes=100 * 1024 * 1024,
        ),
    )(te, xs, W1, W1, W2, W2)


def kernel(x, Wg, W1, W2):
    b, t, d = x.shape
    x_flat = x.reshape(t, d)
    _dispatch, _combine = _sc_kernels()
    s, s0, s1, w0, w1, te = _router_meta(x_flat, Wg)
    xs = _dispatch(x_flat, s.reshape(P_))
    ys = _expert_mlp(te, xs, W1, W2)
    out2 = _combine(ys, s0.reshape(T_), s1.reshape(T_), w0.reshape(T_),
                    w1.reshape(T_))
    out = out2.reshape(b, t, d)
    aux_loss = jnp.asarray(0.0, dtype=x.dtype)
    return (out, aux_loss)


# speedup vs baseline: 1.0409x; 1.0409x over previous
import functools

import jax
import jax.numpy as jnp
from jax import lax
from jax.experimental import pallas as pl
from jax.experimental.pallas import tpu as pltpu
from jax.experimental.pallas import tpu_sc as plsc

T_, D_ = 2048, 768
E_, FF_ = 8, 3072
P_ = 2 * T_
TM = 256
NT_ = P_ // TM + E_
RPAD_ = NT_ * TM
RB_ = 512
LC_ = D_ // 16

NC_, NS_ = 2, 16
NW_ = NC_ * NS_
CH_ = P_ // NW_
TT_ = T_ // NW_

_INV_SQRT2 = 0.7071067811865476


def _router_meta_body(x_ref, wg_ref, s_ref, s0_ref, s1_ref, w0_ref, w1_ref,
                      te_ref):
    xt = x_ref[...]
    logits = lax.dot_general(xt, wg_ref[...], (((1,), (1,)), ((), ())),
                             preferred_element_type=jnp.float32)
    lane = lax.broadcasted_iota(jnp.int32, (T_, E_), 1)
    mx = jnp.max(logits, axis=1, keepdims=True)
    p = jnp.exp(logits - mx)
    p = p / jnp.sum(p, axis=1, keepdims=True)
    m1 = jnp.max(p, axis=1, keepdims=True)
    e1 = jnp.min(jnp.where(p == m1, lane, E_), axis=1, keepdims=True)
    pm = jnp.where(lane == e1, -1.0, p)
    m2 = jnp.max(pm, axis=1, keepdims=True)
    e2 = jnp.min(jnp.where(pm == m2, lane, E_), axis=1, keepdims=True)
    ssum = m1 + m2
    w0_ref[...] = m1 / ssum
    w1_ref[...] = m2 / ssum

    lane_p = lax.broadcasted_iota(jnp.int32, (P_, E_), 1)
    e_pair = jnp.concatenate([e1, e2], axis=0)
    onehot = (e_pair == lane_p).astype(jnp.float32)

    r_i = lax.broadcasted_iota(jnp.int32, (RB_, RB_), 0)
    c_i = lax.broadcasted_iota(jnp.int32, (RB_, RB_), 1)
    ltri = (r_i > c_i).astype(jnp.float32)
    carry = jnp.zeros((1, E_), jnp.float32)
    rank_blocks = []
    for b in range(P_ // RB_):
        mb = onehot[b * RB_:(b + 1) * RB_]
        rb = lax.dot_general(ltri, mb, (((1,), (0,)), ((), ())),
                             preferred_element_type=jnp.float32) + carry
        rank_blocks.append(rb)
        carry = carry + jnp.sum(mb, axis=0, keepdims=True)
    ranks = jnp.concatenate(rank_blocks, axis=0)
    counts = carry

    padded = jnp.floor((counts + (TM - 1)) / TM) * TM
    e8r = lax.broadcasted_iota(jnp.int32, (E_, E_), 0)
    e8c = lax.broadcasted_iota(jnp.int32, (E_, E_), 1)
    ustri = (e8r < e8c).astype(jnp.float32)
    pad_off = lax.dot_general(padded, ustri, (((1,), (0,)), ((), ())),
                              preferred_element_type=jnp.float32)
    slots = jnp.sum(onehot * (ranks + pad_off), axis=1, keepdims=True)
    slots_i = slots.astype(jnp.int32)
    s_ref[...] = slots_i
    s0_ref[...] = slots_i[:T_]
    s1_ref[...] = slots_i[T_:]

    end_t = (pad_off + padded) / TM
    total_t = jnp.sum(padded, axis=1, keepdims=True) / TM
    ti = lax.broadcasted_iota(jnp.int32, (NT_, E_), 0).astype(jnp.float32)
    n_done = jnp.sum(jnp.where(ti >= end_t, 1.0, 0.0), axis=1, keepdims=True)
    expert_col = jnp.minimum(n_done, float(E_ - 1))
    active_col = jnp.where(ti[:, :1] < total_t, 1.0, 0.0)
    te_ref[...] = jnp.concatenate([expert_col, active_col],
                                  axis=1).astype(jnp.int32)


def _router_meta(x_flat, Wg):
    return pl.pallas_call(
        _router_meta_body,
        out_shape=(
            jax.ShapeDtypeStruct((P_, 1), jnp.int32),
            jax.ShapeDtypeStruct((T_, 1), jnp.int32),
            jax.ShapeDtypeStruct((T_, 1), jnp.int32),
            jax.ShapeDtypeStruct((T_, 1), jnp.float32),
            jax.ShapeDtypeStruct((T_, 1), jnp.float32),
            jax.ShapeDtypeStruct((NT_, 2), jnp.int32),
        ),
    )(x_flat, Wg)


@functools.lru_cache(maxsize=None)
def _sc_kernels():
    mesh = plsc.VectorSubcoreMesh(core_axis_name="c", subcore_axis_name="s")

    @functools.partial(
        pl.kernel,
        out_type=jax.ShapeDtypeStruct((RPAD_, D_), jnp.float32),
        mesh=mesh,
        scratch_types=[
            pltpu.VMEM((CH_,), jnp.int32),
            pltpu.VMEM((CH_, D_), jnp.float32),
            pltpu.SemaphoreType.DMA,
        ],
    )
    def _dispatch(x_hbm, s_hbm, xs_hbm, idx_v, rows_v, sem):
        wid = lax.axis_index("s") * NC_ + lax.axis_index("c")
        base = wid * CH_
        tbase = lax.rem(base, T_)
        cp = pltpu.async_copy(x_hbm.at[pl.ds(tbase, CH_)], rows_v, sem)
        pltpu.sync_copy(s_hbm.at[pl.ds(base, CH_)], idx_v)
        cp.wait()
        pltpu.async_copy(rows_v, xs_hbm.at[idx_v], sem).wait()

    @functools.partial(
        pl.kernel,
        out_type=jax.ShapeDtypeStruct((T_, D_), jnp.float32),
        mesh=mesh,
        scratch_types=[
            pltpu.VMEM((TT_,), jnp.int32),
            pltpu.VMEM((TT_,), jnp.int32),
            pltpu.VMEM((TT_ + 16,), jnp.float32),
            pltpu.VMEM((TT_ + 16,), jnp.float32),
            pltpu.VMEM((TT_, D_), jnp.float32),
            pltpu.VMEM((TT_, D_), jnp.float32),
            pltpu.SemaphoreType.DMA,
            pltpu.SemaphoreType.DMA,
        ],
    )
    def _combine(ys_hbm, s0_hbm, s1_hbm, w0_hbm, w1_hbm, out_hbm,
                 i0_v, i1_v, w0_v, w1_v, r0_v, r1_v, sem0, sem1):
        wid = lax.axis_index("s") * NC_ + lax.axis_index("c")
        tb = wid * TT_
        pltpu.sync_copy(s0_hbm.at[pl.ds(tb, TT_)], i0_v)
        pltpu.sync_copy(s1_hbm.at[pl.ds(tb, TT_)], i1_v)
        cp0 = pltpu.async_copy(ys_hbm.at[i0_v], r0_v, sem0)
        cp1 = pltpu.async_copy(ys_hbm.at[i1_v], r1_v, sem1)
        pltpu.sync_copy(w0_hbm.at[pl.ds(tb, TT_)], w0_v.at[pl.ds(0, TT_)])
        pltpu.sync_copy(w1_hbm.at[pl.ds(tb, TT_)], w1_v.at[pl.ds(0, TT_)])
        cp0.wait()
        cp1.wait()

        def body(i, _):
            a = w0_v[pl.ds(i, 16)][0]
            b = w1_v[pl.ds(i, 16)][0]
            for c in range(LC_):
                sl = pl.ds(c * 16, 16)
                r0_v[i, sl] = a * r0_v[i, sl] + b * r1_v[i, sl]
            return 0

        lax.fori_loop(0, TT_, body, 0)
        pltpu.sync_copy(r0_v, out_hbm.at[pl.ds(tb, TT_)])

    return _dispatch, _combine


def _expert_mlp_body(te_ref, xs_ref, w1_ref, w2_ref, ys_ref):
    i = pl.program_id(0)

    @pl.when(te_ref[i, 1] == 1)
    def _():
        h = lax.dot_general(xs_ref[...].astype(jnp.bfloat16),
                            w1_ref[0].astype(jnp.bfloat16),
                            (((1,), (0,)), ((), ())),
                            preferred_element_type=jnp.float32)
        h = 0.5 * h * (1.0 + lax.erf(h * _INV_SQRT2))
        ys_ref[...] = lax.dot_general(h.astype(jnp.bfloat16),
                                      w2_ref[0].astype(jnp.bfloat16),
                                      (((1,), (0,)), ((), ())),
                                      preferred_element_type=jnp.float32)


def _expert_mlp(te, xs, W1, W2):
    grid_spec = pltpu.PrefetchScalarGridSpec(
        num_scalar_prefetch=1,
        grid=(NT_,),
        in_specs=[
            pl.BlockSpec((TM, D_), lambda i, te: (i, 0)),
            pl.BlockSpec((1, D_, FF_), lambda i, te: (te[i, 0], 0, 0)),
            pl.BlockSpec((1, FF_, D_), lambda i, te: (te[i, 0], 0, 0)),
        ],
        out_specs=pl.BlockSpec((TM, D_), lambda i, te: (i, 0)),
    )
    return pl.pallas_call(
        _expert_mlp_body,
        grid_spec=grid_spec,
        out_shape=jax.ShapeDtypeStruct((RPAD_, D_), jnp.float32),
        compiler_params=pltpu.CompilerParams(
            dimension_semantics=("arbitrary",),
        ),
    )(te, xs, W1, W2)


def kernel(x, Wg, W1, W2):
    b, t, d = x.shape
    x_flat = x.reshape(t, d)
    _dispatch, _combine = _sc_kernels()
    s, s0, s1, w0, w1, te = _router_meta(x_flat, Wg)
    xs = _dispatch(x_flat, s.reshape(P_))
    ys = _expert_mlp(te, xs, W1, W2)
    out2 = _combine(ys, s0.reshape(T_), s1.reshape(T_), w0.reshape(T_),
                    w1.reshape(T_))
    out = out2.reshape(b, t, d)
    aux_loss = jnp.asarray(0.0, dtype=x.dtype)
    return (out, aux_loss)
